# baseline (speedup 1.0000x reference)
import dataclasses

import jax
import jax.numpy as jnp
from jax.experimental import pallas as pl
from jax.experimental.pallas import tpu as pltpu
from jax.experimental.pallas import tpu_sc as plsc

_B = 16
_N = 32768
_CH = _N // 32
_NV = _CH // 16
_ROWS = _N // 128

_mesh = plsc.VectorSubcoreMesh(core_axis_name="c", subcore_axis_name="s")

_F = jnp.float32

_cp = pltpu.CompilerParams()
if "needs_layout_passes" in pltpu.CompilerParams.__dataclass_fields__:
    _cp = dataclasses.replace(_cp, needs_layout_passes=False)


def _vfull(v):
    return jnp.full((16,), v, dtype=_F)


def _thr_body(z_ref, seg_ref, thr_ref):
    z = z_ref[...]
    seg = seg_ref[...]
    mns = []
    mxs = []
    for s in range(_B):
        m = seg == s
        mns.append(jnp.min(jnp.where(m, z, jnp.inf)))
        mxs.append(jnp.max(jnp.where(m, z, -jnp.inf)))
    mn = jnp.stack(mns)
    mx = jnp.stack(mxs)
    rng = mx - mn
    zb = mn + 0.405 * rng
    zt = mx - 0.405 * rng
    mid = (mn + mx) / 2.0
    half = 0.19 * (zt - zb) / 2.0
    thr_ref[...] = jnp.stack([zb, zt, mid - half, mid + half])


@jax.named_call
def _sc_main(pred_pl, true_pl, seg, thr):
    @pl.kernel(
        out_type=jax.ShapeDtypeStruct((32, 4, 16, 16), _F),
        mesh=_mesh,
        compiler_params=_cp,
        scratch_types=[
            pltpu.VMEM((_CH,), _F),
            pltpu.VMEM((_CH,), _F),
            pltpu.VMEM((_CH,), _F),
            pltpu.VMEM((_CH,), _F),
            pltpu.VMEM((_CH,), _F),
            pltpu.VMEM((_CH,), _F),
            pltpu.VMEM((_CH,), jnp.int32),
            pltpu.VMEM((4, 16), _F),
            pltpu.VMEM((4, 16, 16), _F),
            pltpu.SemaphoreType.DMA,
            pltpu.SemaphoreType.DMA,
            pltpu.SemaphoreType.DMA,
            pltpu.SemaphoreType.DMA,
            pltpu.SemaphoreType.DMA,
            pltpu.SemaphoreType.DMA,
            pltpu.SemaphoreType.DMA,
            pltpu.SemaphoreType.DMA,
        ],
    )
    def body(pf_hbm, tf_hbm, seg_hbm, thr_hbm, out_hbm,
             px_t, py_t, pz_t, tx_t, ty_t, tz_t, seg_t, thr_t, acc_t,
             sem1, sem2, sem3, sem4, sem5, sem6, sem7, sem8):
        core = jax.lax.axis_index("c")
        sub = jax.lax.axis_index("s")
        tile = core * 16 + sub
        a = tile * _CH

        cp1 = pltpu.async_copy(pf_hbm.at[pl.ds(a, _CH)], px_t, sem1)
        cp2 = pltpu.async_copy(pf_hbm.at[pl.ds(_N + a, _CH)], py_t, sem2)
        cp3 = pltpu.async_copy(pf_hbm.at[pl.ds(2 * _N + a, _CH)], pz_t, sem3)
        cp4 = pltpu.async_copy(tf_hbm.at[pl.ds(a, _CH)], tx_t, sem4)
        cp5 = pltpu.async_copy(tf_hbm.at[pl.ds(_N + a, _CH)], ty_t, sem5)
        cp6 = pltpu.async_copy(tf_hbm.at[pl.ds(2 * _N + a, _CH)], tz_t, sem6)
        cp7 = pltpu.async_copy(seg_hbm.at[pl.ds(a, _CH)], seg_t, sem7)
        cp8 = pltpu.async_copy(thr_hbm, thr_t, sem8)

        lane = jax.lax.iota(jnp.int32, 16)

        def init_acc(s, _):
            acc_t[0, s] = _vfull(0.0)
            acc_t[1, s] = _vfull(0.0)
            acc_t[2, s] = _vfull(0.0)
            acc_t[3, s] = _vfull(0.0)
            return 0

        jax.lax.fori_loop(0, 16, init_acc, 0)

        cp1.wait()
        cp2.wait()
        cp3.wait()
        cp4.wait()
        cp5.wait()
        cp6.wait()
        cp7.wait()
        cp8.wait()

        def hub1(d):
            ad = jnp.abs(d)
            return jnp.where(ad < 0.5, 0.5 * d * d, 0.5 * (ad - 0.25))

        def vec_sum(j, _):
            sl = pl.ds(j * 16, 16)
            sv = seg_t[sl]
            dx = px_t[sl] - tx_t[sl]
            dy = py_t[sl] - ty_t[sl]
            zv = tz_t[sl]
            dz = pz_t[sl] - zv
            hub = hub1(dx) + hub1(dy) + hub1(dz)
            sq = dx * dx + dy * dy + dz * dz
            zbv = plsc.load_gather(thr_t.at[0], [sv])
            ztv = plsc.load_gather(thr_t.at[1], [sv])
            fbv = plsc.load_gather(thr_t.at[2], [sv])
            ftv = plsc.load_gather(thr_t.at[3], [sv])
            fil = (zv >= zbv) & (zv <= ztv) & (zv >= fbv) & (zv <= ftv)
            filf = jnp.where(fil, _vfull(1.0), _vfull(0.0))
            plsc.addupdate_scatter(acc_t.at[0], [sv, lane], filf * hub)
            plsc.addupdate_scatter(acc_t.at[1], [sv, lane], filf)
            plsc.addupdate_scatter(acc_t.at[2], [sv, lane], (1.0 - filf) * sq)
            plsc.addupdate_scatter(acc_t.at[3], [sv, lane], _vfull(1.0))
            return 0

        jax.lax.fori_loop(0, _NV, vec_sum, 0)
        pltpu.sync_copy(acc_t, out_hbm.at[tile])

    return body(pred_pl, true_pl, seg, thr)


def _fin_body(parts_ref, out_ref):
    x = parts_ref[...]
    fs = jnp.sum(x[:, 0], axis=(0, 2))
    fc = jnp.sum(x[:, 1], axis=(0, 2))
    es = jnp.sum(x[:, 2], axis=(0, 2))
    cm = jnp.sum(x[:, 3], axis=(0, 2))
    ec = cm - fc
    zero = jnp.zeros((16,), _F)
    fil_mean = jnp.where(fc > 0, fs / (3.0 * jnp.maximum(fc, 1.0)), zero)
    ele_mean = jnp.where(ec > 0, es / (3.0 * jnp.maximum(ec, 1.0)), zero)
    loss = (50.0 / _B) * jnp.sum(fil_mean) + (1.0 / _B) * jnp.sum(ele_mean)
    out_ref[...] = jnp.reshape(loss, (1, 1))


@jax.jit
def kernel(pred_coords, true_coords, batch_vector):
    pred_pl = pred_coords.T.reshape(3 * _N)
    true_pl = true_coords.T.reshape(3 * _N)
    seg = batch_vector.astype(jnp.int32)
    thr = pl.pallas_call(
        _thr_body,
        out_shape=jax.ShapeDtypeStruct((4, _B), _F),
    )(true_pl[2 * _N:].reshape(_ROWS, 128), seg.reshape(_ROWS, 128))
    parts = _sc_main(pred_pl, true_pl, seg, thr)
    out = pl.pallas_call(
        _fin_body,
        out_shape=jax.ShapeDtypeStruct((1, 1), _F),
    )(parts)
    return out[0, 0]


# speedup vs baseline: 1.0088x; 1.0088x over previous
import dataclasses

import jax
import jax.numpy as jnp
from jax.experimental import pallas as pl
from jax.experimental.pallas import tpu as pltpu
from jax.experimental.pallas import tpu_sc as plsc

_B = 16
_N = 32768
_NT = 16
_CH = _N // _NT
_NV = _CH // 16

_mesh = plsc.VectorSubcoreMesh(core_axis_name="c", subcore_axis_name="s")

_F = jnp.float32

_cp = pltpu.CompilerParams()
if "needs_layout_passes" in pltpu.CompilerParams.__dataclass_fields__:
    _cp = dataclasses.replace(_cp, needs_layout_passes=False)


def _vfull(v):
    return jnp.full((16,), v, dtype=_F)


@jax.named_call
def _sc_loss(pred_pl, true_pl, seg):
    @pl.kernel(
        out_type=(jax.ShapeDtypeStruct((16,), _F),
                  jax.ShapeDtypeStruct((2, 16, 16), _F),
                  jax.ShapeDtypeStruct((_NT, 4, 16), _F)),
        mesh=_mesh,
        compiler_params=_cp,
        scratch_types=[
            pltpu.VMEM((_CH,), _F),
            pltpu.VMEM((_CH,), _F),
            pltpu.VMEM((_CH,), _F),
            pltpu.VMEM((_CH,), _F),
            pltpu.VMEM((_CH,), _F),
            pltpu.VMEM((_CH,), _F),
            pltpu.VMEM((_CH,), jnp.int32),
            pltpu.VMEM((2, 16, 16), _F),
            pltpu.VMEM((2, 16, 16), _F),
            pltpu.VMEM((4, 16), _F),
            pltpu.VMEM((4, 16, 16), _F),
            pltpu.VMEM((4, 16), _F),
            pltpu.VMEM((_NT, 4, 16), _F),
            pltpu.VMEM((16,), _F),
            pltpu.SemaphoreType.DMA,
            pltpu.SemaphoreType.DMA,
            pltpu.SemaphoreType.DMA,
            pltpu.SemaphoreType.DMA,
            pltpu.SemaphoreType.DMA,
            pltpu.SemaphoreType.DMA,
            pltpu.SemaphoreType.DMA,
        ],
    )
    def body(pf_hbm, tf_hbm, seg_hbm, out_hbm, xmm_hbm, xacc_hbm,
             px_t, py_t, pz_t, tx_t, ty_t, tz_t, seg_t,
             mm2_t, mm_t, thr_t, acc_t, acc2_t, part_t, res_t,
             sem1, sem2, sem3, sem4, sem5, sem6, sem7):
        core = jax.lax.axis_index("c")
        sub = jax.lax.axis_index("s")

        def main(_):
            a = sub * _CH
            cp1 = pltpu.async_copy(tf_hbm.at[pl.ds(2 * _N + a, _CH)], tz_t,
                                   sem1)
            cp2 = pltpu.async_copy(seg_hbm.at[pl.ds(a, _CH)], seg_t, sem2)
            cp3 = pltpu.async_copy(pf_hbm.at[pl.ds(a, _CH)], px_t, sem3)
            cp4 = pltpu.async_copy(pf_hbm.at[pl.ds(_N + a, _CH)], py_t, sem4)
            cp5 = pltpu.async_copy(pf_hbm.at[pl.ds(2 * _N + a, _CH)], pz_t,
                                   sem5)
            cp6 = pltpu.async_copy(tf_hbm.at[pl.ds(a, _CH)], tx_t, sem6)
            cp7 = pltpu.async_copy(tf_hbm.at[pl.ds(_N + a, _CH)], ty_t, sem7)

            lane = jax.lax.iota(jnp.int32, 16)

            def init_mm(s, _):
                mm2_t[0, s] = _vfull(jnp.inf)
                mm2_t[1, s] = _vfull(-jnp.inf)
                return 0

            jax.lax.fori_loop(0, 16, init_mm, 0)
            cp1.wait()
            cp2.wait()

            def vec_mm(i, _):
                sl = pl.ds(i * 16, 16)
                zv = tz_t[sl]
                sv = seg_t[sl]
                old_mn = plsc.load_gather(mm2_t.at[0], [sv, lane])
                plsc.store_scatter(mm2_t.at[0], [sv, lane],
                                   jnp.minimum(old_mn, zv))
                old_mx = plsc.load_gather(mm2_t.at[1], [sv, lane])
                plsc.store_scatter(mm2_t.at[1], [sv, lane],
                                   jnp.maximum(old_mx, zv))
                return 0

            jax.lax.fori_loop(0, _NV, vec_mm, 0)

            def row_red(s, c):
                rmin, rmax = c
                rmin = jnp.where(lane == s,
                                 jnp.broadcast_to(jnp.min(mm2_t[0, s]), (16,)),
                                 rmin)
                rmax = jnp.where(lane == s,
                                 jnp.broadcast_to(jnp.max(mm2_t[1, s]), (16,)),
                                 rmax)
                return rmin, rmax

            rmin, rmax = jax.lax.fori_loop(
                0, 16, row_red, (_vfull(jnp.inf), _vfull(-jnp.inf)))

            thr_t[0] = rmin
            thr_t[1] = rmax
            cpa = pltpu.async_copy(thr_t.at[0], xmm_hbm.at[0, sub], sem1)
            cpb = pltpu.async_copy(thr_t.at[1], xmm_hbm.at[1, sub], sem2)
            cpa.wait()
            cpb.wait()
            plsc.subcore_barrier()
            pltpu.sync_copy(xmm_hbm, mm_t)

            def red_mm(k, carry):
                mn, mx = carry
                return (jnp.minimum(mn, mm_t[0, k]), jnp.maximum(mx, mm_t[1, k]))

            mn, mx = jax.lax.fori_loop(0, 16, red_mm,
                                       (_vfull(jnp.inf), _vfull(-jnp.inf)))
            rng = mx - mn
            zb = mn + 0.405 * rng
            zt = mx - 0.405 * rng
            mid = (mn + mx) / 2.0
            half = 0.19 * (zt - zb) / 2.0
            thr_t[0] = zb
            thr_t[1] = zt
            thr_t[2] = mid - half
            thr_t[3] = mid + half

            def init_acc(s, _):
                acc_t[0, s] = _vfull(0.0)
                acc_t[1, s] = _vfull(0.0)
                acc_t[2, s] = _vfull(0.0)
                acc_t[3, s] = _vfull(0.0)
                return 0

            jax.lax.fori_loop(0, 16, init_acc, 0)
            cp3.wait()
            cp4.wait()
            cp5.wait()
            cp6.wait()
            cp7.wait()

            def hub1(d):
                ad = jnp.abs(d)
                return jnp.where(ad < 0.5, 0.5 * d * d, 0.5 * (ad - 0.25))

            def vec_sum(j, _):
                sl = pl.ds(j * 16, 16)
                sv = seg_t[sl]
                dx = px_t[sl] - tx_t[sl]
                dy = py_t[sl] - ty_t[sl]
                zv = tz_t[sl]
                dz = pz_t[sl] - zv
                hub = hub1(dx) + hub1(dy) + hub1(dz)
                sq = dx * dx + dy * dy + dz * dz
                zbv = plsc.load_gather(thr_t.at[0], [sv])
                ztv = plsc.load_gather(thr_t.at[1], [sv])
                fbv = plsc.load_gather(thr_t.at[2], [sv])
                ftv = plsc.load_gather(thr_t.at[3], [sv])
                fil = (zv >= zbv) & (zv <= ztv) & (zv >= fbv) & (zv <= ftv)
                filf = jnp.where(fil, _vfull(1.0), _vfull(0.0))
                plsc.addupdate_scatter(acc_t.at[0], [sv, lane], filf * hub)
                plsc.addupdate_scatter(acc_t.at[1], [sv, lane], filf)
                plsc.addupdate_scatter(acc_t.at[2], [sv, lane],
                                       (1.0 - filf) * sq)
                plsc.addupdate_scatter(acc_t.at[3], [sv, lane], _vfull(1.0))
                return 0

            jax.lax.fori_loop(0, _NV, vec_sum, 0)

            def lane_red(s, c):
                a0, a1, a2, a3 = c
                here = lane == s
                a0 = jnp.where(here,
                               jnp.broadcast_to(jnp.sum(acc_t[0, s]), (16,)), a0)
                a1 = jnp.where(here,
                               jnp.broadcast_to(jnp.sum(acc_t[1, s]), (16,)), a1)
                a2 = jnp.where(here,
                               jnp.broadcast_to(jnp.sum(acc_t[2, s]), (16,)), a2)
                a3 = jnp.where(here,
                               jnp.broadcast_to(jnp.sum(acc_t[3, s]), (16,)), a3)
                return a0, a1, a2, a3

            z16 = _vfull(0.0)
            a0, a1, a2, a3 = jax.lax.fori_loop(0, 16, lane_red,
                                               (z16, z16, z16, z16))
            acc2_t[0] = a0
            acc2_t[1] = a1
            acc2_t[2] = a2
            acc2_t[3] = a3
            cpc = pltpu.async_copy(acc2_t, xacc_hbm.at[sub], sem1)
            cpc.wait()
            plsc.subcore_barrier()

            def do_final(_):
                pltpu.sync_copy(xacc_hbm, part_t)
                z16 = _vfull(0.0)

                def tile_red(k, c):
                    fs, fc, es, cm = c
                    return (fs + part_t[k, 0], fc + part_t[k, 1],
                            es + part_t[k, 2], cm + part_t[k, 3])

                fs, fc, es, cm = jax.lax.fori_loop(0, _NT, tile_red,
                                                   (z16, z16, z16, z16))
                ec = cm - fc
                fil_mean = jnp.where(fc > 0, fs / (3.0 * jnp.maximum(fc, 1.0)),
                                     z16)
                ele_mean = jnp.where(ec > 0, es / (3.0 * jnp.maximum(ec, 1.0)),
                                     z16)
                loss = ((50.0 / _B) * jnp.sum(fil_mean)
                        + (1.0 / _B) * jnp.sum(ele_mean))
                res_t[...] = jnp.broadcast_to(loss, (16,))
                pltpu.sync_copy(res_t, out_hbm)
                return 0

            jax.lax.cond(sub == 0, do_final, lambda _: 0, 0)
            return 0

        jax.lax.cond(core == 0, main, lambda _: 0, 0)

    return body(pred_pl, true_pl, seg)


@jax.jit
def kernel(pred_coords, true_coords, batch_vector):
    pred_pl = pred_coords.T.reshape(3 * _N)
    true_pl = true_coords.T.reshape(3 * _N)
    seg = batch_vector.astype(jnp.int32)
    out, _, _ = _sc_loss(pred_pl, true_pl, seg)
    return out[0]
